# Initial kernel scaffold; baseline (speedup 1.0000x reference)
import jax
import jax.numpy as jnp
from jax.experimental import pallas as pl


def kernel(x, edge_index, W1, b1, W2, b2, Wout, bout):
    raise NotImplementedError("write your pallas kernel here")


# speedup vs baseline: 26.2475x; 26.2475x over previous
import functools

import jax
import jax.numpy as jnp
from jax import lax
from jax.experimental import pallas as pl
from jax.experimental.pallas import tpu as pltpu
from jax.experimental.pallas import tpu_sc as plsc

N = 10000
E = 320000
HID = 32
NC, NS = 2, 16
NW = NC * NS
CHUNK = 128
CPW = -(-E // (NW * CHUNK))
EPW = CPW * CHUNK
E_PAD = NW * EPW
NPAD = 10240
RPT = NPAD // NS
SINK = N

_mesh = plsc.VectorSubcoreMesh(core_axis_name="c", subcore_axis_name="s")
_sc_params = pltpu.CompilerParams(use_tc_tiling_on_sc=False)


@functools.partial(
    pl.kernel,
    out_type=jax.ShapeDtypeStruct((NC, NPAD, 16), jnp.float32),
    mesh=_mesh,
    scratch_types=[
        pltpu.VMEM((CPW, CHUNK), jnp.int32),
        pltpu.VMEM((CHUNK, 16), jnp.float32),
        pltpu.VMEM_SHARED((NPAD, 16), jnp.float32),
    ],
    compiler_params=_sc_params,
)
def _deg_kernel(dst_hbm, out_hbm, dstbuf, buf, deg_sh):
    c = lax.axis_index("c")
    s = lax.axis_index("s")
    wid = s * NC + c
    pltpu.sync_copy(dst_hbm.at[wid], dstbuf)

    def _zfill(i, _):
        buf[i, :] = jnp.zeros((16,), jnp.float32)
        return 0
    lax.fori_loop(0, CHUNK, _zfill, 0)
    for k in range(RPT // CHUNK):
        pltpu.sync_copy(buf, deg_sh.at[pl.ds(s * RPT + k * CHUNK, CHUNK)])

    def _ofill(i, _):
        buf[i, :] = jnp.ones((16,), jnp.float32)
        return 0
    lax.fori_loop(0, CHUNK, _ofill, 0)
    plsc.subcore_barrier()

    def _scat(j, _):
        pltpu.sync_copy(buf, deg_sh.at[dstbuf.at[j]], add=True)
        return 0
    lax.fori_loop(0, CPW, _scat, 0)
    plsc.subcore_barrier()
    pltpu.sync_copy(deg_sh.at[pl.ds(s * RPT, RPT)],
                    out_hbm.at[c, pl.ds(s * RPT, RPT)])


@functools.partial(
    pl.kernel,
    out_type=jax.ShapeDtypeStruct((NC, NPAD, HID), jnp.float32),
    mesh=_mesh,
    scratch_types=[
        pltpu.VMEM((CPW, CHUNK), jnp.int32),
        pltpu.VMEM((CPW, CHUNK), jnp.int32),
        pltpu.VMEM((CHUNK, HID), jnp.float32),
        pltpu.VMEM((CHUNK, HID), jnp.float32),
        pltpu.VMEM_SHARED((NPAD, HID), jnp.float32),
        pltpu.SemaphoreType.DMA,
    ],
    compiler_params=_sc_params,
)
def _agg_kernel(y_hbm, src_hbm, dst_hbm, out_hbm,
                srcbuf, dstbuf, rows, zbuf, agg_sh, sem):
    c = lax.axis_index("c")
    s = lax.axis_index("s")
    wid = s * NC + c
    pltpu.sync_copy(src_hbm.at[wid], srcbuf)
    pltpu.sync_copy(dst_hbm.at[wid], dstbuf)

    def _zfill(i, _):
        zbuf[i, pl.ds(0, 16)] = jnp.zeros((16,), jnp.float32)
        zbuf[i, pl.ds(16, 16)] = jnp.zeros((16,), jnp.float32)
        return 0
    lax.fori_loop(0, CHUNK, _zfill, 0)
    for k in range(RPT // CHUNK):
        pltpu.sync_copy(zbuf, agg_sh.at[pl.ds(s * RPT + k * CHUNK, CHUNK)])
    plsc.subcore_barrier()

    def _step(j, _):
        pltpu.async_copy(y_hbm.at[srcbuf.at[j]], rows, sem).wait()
        pltpu.sync_copy(rows, agg_sh.at[dstbuf.at[j]], add=True)
        return 0
    lax.fori_loop(0, CPW, _step, 0)
    plsc.subcore_barrier()
    pltpu.sync_copy(agg_sh.at[pl.ds(s * RPT, RPT)],
                    out_hbm.at[c, pl.ds(s * RPT, RPT)])


def _tc_a_body(x_ref, w1_ref, d0_ref, d1_ref, y1_ref, dinv_ref):
    deg = d0_ref[...] + d1_ref[...] + 1.0
    dinv = lax.rsqrt(deg)
    xw = jnp.dot(x_ref[...], w1_ref[...], preferred_element_type=jnp.float32)
    y1_ref[...] = xw * dinv
    dinv_ref[...] = dinv


_tc_a = pl.pallas_call(
    _tc_a_body,
    out_shape=[jax.ShapeDtypeStruct((N, HID), jnp.float32),
               jax.ShapeDtypeStruct((N, 1), jnp.float32)],
)


def _tc_b_body(p0_ref, p1_ref, y1_ref, dinv_ref, b1_ref, w2_ref, y2_ref):
    dinv = dinv_ref[...]
    pre = dinv * (p0_ref[...] + p1_ref[...] + y1_ref[...]) + b1_ref[...]
    h1 = jnp.maximum(pre, 0.0)
    xw2 = jnp.dot(h1, w2_ref[...], preferred_element_type=jnp.float32)
    y2_ref[...] = xw2 * dinv


_tc_b = pl.pallas_call(
    _tc_b_body,
    out_shape=jax.ShapeDtypeStruct((N, HID), jnp.float32),
)


def _tc_c_body(p0_ref, p1_ref, y2_ref, dinv_ref, b2_ref, wout_ref, bout_ref,
               out_ref):
    dinv = dinv_ref[...]
    pre = dinv * (p0_ref[...] + p1_ref[...] + y2_ref[...]) + b2_ref[...]
    h2 = jnp.maximum(pre, 0.0)
    z = jnp.dot(h2, wout_ref[...], preferred_element_type=jnp.float32)
    out_ref[...] = jax.nn.sigmoid(z + bout_ref[...])


_tc_c = pl.pallas_call(
    _tc_c_body,
    out_shape=jax.ShapeDtypeStruct((N, 1), jnp.float32),
)


def kernel(x, edge_index, W1, b1, W2, b2, Wout, bout):
    ei = edge_index.astype(jnp.int32)
    src = jnp.concatenate([ei[0], jnp.zeros((E_PAD - E,), jnp.int32)])
    dst = jnp.concatenate([ei[1], jnp.full((E_PAD - E,), SINK, jnp.int32)])
    src3 = src.reshape(NW, CPW, CHUNK)
    dst3 = dst.reshape(NW, CPW, CHUNK)

    degp = _deg_kernel(dst3)
    d0 = degp[0, :N, 0:1]
    d1 = degp[1, :N, 0:1]
    y1, dinv = _tc_a(x, W1, d0, d1)

    p1 = _agg_kernel(y1, src3, dst3)
    y2 = _tc_b(p1[0, :N], p1[1, :N], y1, dinv, b1.reshape(1, HID), W2)

    p2 = _agg_kernel(y2, src3, dst3)
    out = _tc_c(p2[0, :N], p2[1, :N], y2, dinv, b2.reshape(1, HID),
                Wout, bout.reshape(1, 1))
    return out
